# Initial kernel scaffold; baseline (speedup 1.0000x reference)
import jax
import jax.numpy as jnp
from jax.experimental import pallas as pl


def kernel(xyz, new_xyz, features):
    raise NotImplementedError("write your pallas kernel here")


# speedup vs baseline: 6.7989x; 6.7989x over previous
import functools

import jax
import jax.numpy as jnp
from jax import lax
from jax.experimental import pallas as pl
from jax.experimental.pallas import tpu as pltpu
from jax.experimental.pallas import tpu_sc as plsc

K = 32
B = 4
M = 2048
N = 8192
C_XYZ = 6
C_FEAT = 64
BM = 256

NW = 32
QW = (B * M) // NW
ROWS_W = QW * K
FCHUNK = 1024


def _topk_body(low_ref, high_ref, vals_ref, idx_ref, idxabs_ref):
    b = pl.program_id(0)
    low = low_ref[0]
    high = high_ref[0]
    dx = low[:, 0:1] - high[:, 0][None, :]
    dy = low[:, 1:2] - high[:, 1][None, :]
    dz = low[:, 2:3] - high[:, 2][None, :]
    dist = jnp.sqrt((dx * dx + dy * dy) + dz * dz)
    iota_n = lax.broadcasted_iota(jnp.int32, (BM, N), 1)
    col_k = lax.broadcasted_iota(jnp.int32, (BM, K), 1)

    def step(k, carry):
        d, vals, idxs = carry
        m = jnp.min(d, axis=1)
        cand = jnp.where(d == m[:, None], iota_n, N)
        i = jnp.min(cand, axis=1)
        vals = jnp.where(col_k == k, m[:, None], vals)
        idxs = jnp.where(col_k == k, i[:, None], idxs)
        d = jnp.where(iota_n == i[:, None], jnp.inf, d)
        return d, vals, idxs

    vals0 = jnp.zeros((BM, K), jnp.float32)
    idxs0 = jnp.zeros((BM, K), jnp.int32)
    _, vals, idxs = lax.fori_loop(0, K, step, (dist, vals0, idxs0))
    vals_ref[0] = vals
    idx_ref[0] = idxs
    idxabs_ref[0] = idxs + b * N


def _topk(new_xyz, xyz):
    grid = (B, M // BM)
    return pl.pallas_call(
        _topk_body,
        grid=grid,
        in_specs=[
            pl.BlockSpec((1, BM, C_XYZ), lambda b, m: (b, m, 0)),
            pl.BlockSpec((1, N, C_XYZ), lambda b, m: (b, 0, 0)),
        ],
        out_specs=[
            pl.BlockSpec((1, BM, K), lambda b, m: (b, m, 0)),
            pl.BlockSpec((1, BM, K), lambda b, m: (b, m, 0)),
            pl.BlockSpec((1, BM, K), lambda b, m: (b, m, 0)),
        ],
        out_shape=[
            jax.ShapeDtypeStruct((B, M, K), jnp.float32),
            jax.ShapeDtypeStruct((B, M, K), jnp.int32),
            jax.ShapeDtypeStruct((B, M, K), jnp.int32),
        ],
    )(new_xyz, xyz)


def _make_xyz_gather():
    mesh = plsc.VectorSubcoreMesh(core_axis_name="c", subcore_axis_name="s")

    @functools.partial(
        pl.kernel,
        mesh=mesh,
        compiler_params=pltpu.CompilerParams(needs_layout_passes=False),
        out_type=jax.ShapeDtypeStruct((NW * ROWS_W * C_XYZ,), jnp.float32),
        scratch_types=[
            pltpu.VMEM((N * C_XYZ,), jnp.float32),
            pltpu.VMEM((ROWS_W,), jnp.int32),
            pltpu.VMEM((ROWS_W * C_XYZ,), jnp.float32),
        ],
    )
    def xyz_gather(xyz_hbm, idx_hbm, out_hbm, table_v, idx_v, rows_v):
        wid = lax.axis_index("s") * 2 + lax.axis_index("c")
        batch = wid // (NW // B)
        base = wid * ROWS_W
        pltpu.sync_copy(xyz_hbm.at[batch], table_v)
        pltpu.sync_copy(idx_hbm.at[pl.ds(base, ROWS_W)], idx_v)
        lane = lax.broadcasted_iota(jnp.int32, (16,), 0)

        def body(g, carry):
            iv = idx_v[pl.ds(g * 16, 16)] * C_XYZ
            dst16 = (g * 16 + lane) * C_XYZ
            for c in range(C_XYZ):
                vals = plsc.load_gather(table_v, [iv + c])
                plsc.store_scatter(rows_v, [dst16 + c], vals)
            return carry

        lax.fori_loop(0, ROWS_W // 16, body, 0)
        pltpu.sync_copy(rows_v, out_hbm.at[pl.ds(base * C_XYZ, ROWS_W * C_XYZ)])

    return xyz_gather


def _make_feat_gather():
    mesh = plsc.VectorSubcoreMesh(core_axis_name="c", subcore_axis_name="s")

    @functools.partial(
        pl.kernel,
        mesh=mesh,
        compiler_params=pltpu.CompilerParams(
            needs_layout_passes=False, use_tc_tiling_on_sc=False),
        out_type=jax.ShapeDtypeStruct((NW * ROWS_W, C_FEAT), jnp.float32),
        scratch_types=[
            pltpu.VMEM((FCHUNK,), jnp.int32),
            pltpu.VMEM((FCHUNK, C_FEAT), jnp.float32),
            pltpu.SemaphoreType.DMA,
        ],
    )
    def feat_gather(feat_hbm, idxabs_hbm, out_hbm, idx_v, rows_v, sem):
        wid = lax.axis_index("s") * 2 + lax.axis_index("c")
        base = wid * ROWS_W
        for ch in range(ROWS_W // FCHUNK):
            off = base + ch * FCHUNK
            pltpu.sync_copy(idxabs_hbm.at[pl.ds(off, FCHUNK)], idx_v)
            pltpu.async_copy(feat_hbm.at[idx_v], rows_v, sem).wait()
            pltpu.sync_copy(rows_v, out_hbm.at[pl.ds(off, FCHUNK)])

    return feat_gather


def kernel(xyz, new_xyz, features):
    vals, idxs, idxs_abs = _topk(new_xyz, xyz)
    idx_flat = idxs.reshape(B * M * K)
    idxabs_flat = idxs_abs.reshape(B * M * K)
    feat_flat = features.reshape(B * N, C_FEAT)

    neighbor_xyz = _make_xyz_gather()(xyz.reshape(B, N * C_XYZ), idx_flat)
    neighbor_feature = _make_feat_gather()(feat_flat, idxabs_flat)

    return (
        neighbor_xyz.reshape(B, M, K, C_XYZ),
        idxs,
        neighbor_feature.reshape(B, M, K, C_FEAT),
        vals,
    )


# speedup vs baseline: 9.0154x; 1.3260x over previous
import functools

import jax
import jax.numpy as jnp
from jax import lax
from jax.experimental import pallas as pl
from jax.experimental.pallas import tpu as pltpu
from jax.experimental.pallas import tpu_sc as plsc

K = 32
B = 4
M = 2048
N = 8192
C_XYZ = 6
C_FEAT = 64
BM = 256

NW = 32
RPW = (B * M) // NW
ROWS_W = RPW * K
FCHUNK = 1024
NV = N // 16
CAP = 2048
INF = float("inf")


def _dist_body(low_ref, high_ref, dist_ref):
    low = low_ref[0]
    high = high_ref[0]
    dx = low[:, 0:1] - high[:, 0][None, :]
    dy = low[:, 1:2] - high[:, 1][None, :]
    dz = low[:, 2:3] - high[:, 2][None, :]
    dist_ref[0] = jnp.sqrt((dx * dx + dy * dy) + dz * dz)


def _dist_map(new_xyz, xyz):
    return pl.pallas_call(
        _dist_body,
        grid=(B, M // BM),
        in_specs=[
            pl.BlockSpec((1, BM, C_XYZ), lambda b, m: (b, m, 0)),
            pl.BlockSpec((1, N, C_XYZ), lambda b, m: (b, 0, 0)),
        ],
        out_specs=pl.BlockSpec((1, BM, N), lambda b, m: (b, m, 0)),
        out_shape=jax.ShapeDtypeStruct((B, M, N), jnp.float32),
    )(new_xyz, xyz)


def _make_select():
    mesh = plsc.VectorSubcoreMesh(core_axis_name="c", subcore_axis_name="s")

    @functools.partial(
        pl.kernel,
        mesh=mesh,
        compiler_params=pltpu.CompilerParams(needs_layout_passes=False),
        out_type=[
            jax.ShapeDtypeStruct((B * M * K,), jnp.float32),
            jax.ShapeDtypeStruct((B * M * K,), jnp.int32),
            jax.ShapeDtypeStruct((B * M * K,), jnp.int32),
        ],
        scratch_types=[
            pltpu.VMEM((N,), jnp.float32),
            pltpu.VMEM((N,), jnp.float32),
            pltpu.VMEM((CAP + 16,), jnp.float32),
            pltpu.VMEM((CAP + 16,), jnp.int32),
            pltpu.VMEM((RPW * K,), jnp.float32),
            pltpu.VMEM((RPW * K,), jnp.int32),
            pltpu.VMEM((RPW * K,), jnp.int32),
            pltpu.SemaphoreType.DMA,
            pltpu.SemaphoreType.DMA,
        ],
    )
    def select(dist_hbm, vals_hbm, idx_hbm, idxabs_hbm,
               row0_v, row1_v, cval_v, cidx_v, ov_v, oi_v, oa_v, sem0, sem1):
        wid = lax.axis_index("s") * 2 + lax.axis_index("c")
        base = wid * RPW
        lane = lax.broadcasted_iota(jnp.int32, (16,), 0)
        rows = (row0_v, row1_v)
        sems = (sem0, sem1)

        pltpu.async_copy(dist_hbm.at[pl.ds(base * N, N)], row0_v, sem0)

        def process(r, row_v):
            acc0 = row_v[pl.ds(0, 16)]
            acc1 = row_v[pl.ds(16, 16)]

            def gstep(p, carry):
                a0, a1 = carry
                a0 = jnp.minimum(a0, row_v[pl.ds((2 * p) * 16, 16)])
                a1 = jnp.minimum(a1, row_v[pl.ds((2 * p + 1) * 16, 16)])
                return a0, a1

            acc0, acc1 = lax.fori_loop(1, NV // 2, gstep, (acc0, acc1))
            u = jnp.max(jnp.maximum(acc0, acc1))
            u_spl = jnp.full((16,), u, jnp.float32)

            def cstep(g, off):
                v = row_v[pl.ds(g * 16, 16)]
                msk = v <= u_spl
                offc = jnp.minimum(off, CAP)
                plsc.store_compressed(cval_v.at[pl.ds(offc, 16)], v, mask=msk)
                plsc.store_compressed(cidx_v.at[pl.ds(offc, 16)],
                                      g * 16 + lane, mask=msk)
                return off + jnp.sum(msk.astype(jnp.int32))

            off = lax.fori_loop(0, NV, cstep, jnp.int32(0))
            off = jnp.minimum(off, CAP)
            cval_v[pl.ds(off, 16)] = jnp.full((16,), INF)
            nv = (off + 15) // 16

            boff = ((base + r) // M) * N

            def kstep(k, carry):
                av0, av1, ai0, ai1 = carry

                def mstep(g, a):
                    return jnp.minimum(a, cval_v[pl.ds(g * 16, 16)])

                mv = lax.fori_loop(0, nv, mstep, jnp.full((16,), INF))
                m = jnp.min(mv)
                m_spl = jnp.full((16,), m, jnp.float32)

                def istep(g, a):
                    v = cval_v[pl.ds(g * 16, 16)]
                    iv = cidx_v[pl.ds(g * 16, 16)]
                    return jnp.minimum(a, jnp.where(v == m_spl, iv, N))

                i = jnp.min(lax.fori_loop(0, nv, istep, jnp.full((16,), N,
                                                                 jnp.int32)))
                i_spl = jnp.full((16,), i, jnp.int32)

                def wstep(g, a):
                    v = cval_v[pl.ds(g * 16, 16)]
                    iv = cidx_v[pl.ds(g * 16, 16)]
                    cval_v[pl.ds(g * 16, 16)] = jnp.where(iv == i_spl, INF, v)
                    return a

                lax.fori_loop(0, nv, wstep, 0)

                sel_lo = lane == (k % 16)
                hi = k // 16
                av0 = jnp.where(sel_lo & (hi == 0), m, av0)
                av1 = jnp.where(sel_lo & (hi == 1), m, av1)
                ai0 = jnp.where(sel_lo & (hi == 0), i, ai0)
                ai1 = jnp.where(sel_lo & (hi == 1), i, ai1)
                return av0, av1, ai0, ai1

            z_f = jnp.zeros((16,), jnp.float32)
            z_i = jnp.zeros((16,), jnp.int32)
            av0, av1, ai0, ai1 = lax.fori_loop(0, K, kstep,
                                               (z_f, z_f, z_i, z_i))
            ov_v[pl.ds(r * K, 16)] = av0
            ov_v[pl.ds(r * K + 16, 16)] = av1
            oi_v[pl.ds(r * K, 16)] = ai0
            oi_v[pl.ds(r * K + 16, 16)] = ai1
            oa_v[pl.ds(r * K, 16)] = ai0 + boff
            oa_v[pl.ds(r * K + 16, 16)] = ai1 + boff

        def pair(g2, carry):
            for bsel in range(2):
                r = g2 * 2 + bsel
                pltpu.make_async_copy(
                    dist_hbm.at[pl.ds((base + r) * N, N)],
                    rows[bsel], sems[bsel]).wait()

                @pl.when(r + 1 < RPW)
                def _():
                    pltpu.async_copy(
                        dist_hbm.at[pl.ds((base + r + 1) * N, N)],
                        rows[1 - bsel], sems[1 - bsel])

                process(r, rows[bsel])
            return carry

        lax.fori_loop(0, RPW // 2, pair, 0)
        pltpu.sync_copy(ov_v, vals_hbm.at[pl.ds(base * K, RPW * K)])
        pltpu.sync_copy(oi_v, idx_hbm.at[pl.ds(base * K, RPW * K)])
        pltpu.sync_copy(oa_v, idxabs_hbm.at[pl.ds(base * K, RPW * K)])

    return select


def _make_xyz_gather():
    mesh = plsc.VectorSubcoreMesh(core_axis_name="c", subcore_axis_name="s")

    @functools.partial(
        pl.kernel,
        mesh=mesh,
        compiler_params=pltpu.CompilerParams(needs_layout_passes=False),
        out_type=jax.ShapeDtypeStruct((NW * ROWS_W * C_XYZ,), jnp.float32),
        scratch_types=[
            pltpu.VMEM((N * C_XYZ,), jnp.float32),
            pltpu.VMEM((ROWS_W,), jnp.int32),
            pltpu.VMEM((ROWS_W * C_XYZ,), jnp.float32),
        ],
    )
    def xyz_gather(xyz_hbm, idx_hbm, out_hbm, table_v, idx_v, rows_v):
        wid = lax.axis_index("s") * 2 + lax.axis_index("c")
        batch = wid // (NW // B)
        base = wid * ROWS_W
        pltpu.sync_copy(xyz_hbm.at[batch], table_v)
        pltpu.sync_copy(idx_hbm.at[pl.ds(base, ROWS_W)], idx_v)
        lane = lax.broadcasted_iota(jnp.int32, (16,), 0)

        def body(g, carry):
            iv = idx_v[pl.ds(g * 16, 16)] * C_XYZ
            dst16 = (g * 16 + lane) * C_XYZ
            for c in range(C_XYZ):
                vals = plsc.load_gather(table_v, [iv + c])
                plsc.store_scatter(rows_v, [dst16 + c], vals)
            return carry

        lax.fori_loop(0, ROWS_W // 16, body, 0)
        pltpu.sync_copy(rows_v, out_hbm.at[pl.ds(base * C_XYZ, ROWS_W * C_XYZ)])

    return xyz_gather


def _make_feat_gather():
    mesh = plsc.VectorSubcoreMesh(core_axis_name="c", subcore_axis_name="s")

    @functools.partial(
        pl.kernel,
        mesh=mesh,
        compiler_params=pltpu.CompilerParams(
            needs_layout_passes=False, use_tc_tiling_on_sc=False),
        out_type=jax.ShapeDtypeStruct((NW * ROWS_W, C_FEAT), jnp.float32),
        scratch_types=[
            pltpu.VMEM((FCHUNK,), jnp.int32),
            pltpu.VMEM((FCHUNK, C_FEAT), jnp.float32),
            pltpu.SemaphoreType.DMA,
        ],
    )
    def feat_gather(feat_hbm, idxabs_hbm, out_hbm, idx_v, rows_v, sem):
        wid = lax.axis_index("s") * 2 + lax.axis_index("c")
        base = wid * ROWS_W
        for ch in range(ROWS_W // FCHUNK):
            off = base + ch * FCHUNK
            pltpu.sync_copy(idxabs_hbm.at[pl.ds(off, FCHUNK)], idx_v)
            pltpu.async_copy(feat_hbm.at[idx_v], rows_v, sem).wait()
            pltpu.sync_copy(rows_v, out_hbm.at[pl.ds(off, FCHUNK)])

    return feat_gather


def kernel(xyz, new_xyz, features):
    dist = _dist_map(new_xyz, xyz)
    vals_f, idx_f, idxabs_f = _make_select()(dist.reshape(B * M * N))
    feat_flat = features.reshape(B * N, C_FEAT)

    neighbor_xyz = _make_xyz_gather()(xyz.reshape(B, N * C_XYZ), idx_f)
    neighbor_feature = _make_feat_gather()(feat_flat, idxabs_f)

    return (
        neighbor_xyz.reshape(B, M, K, C_XYZ),
        idx_f.reshape(B, M, K),
        neighbor_feature.reshape(B, M, K, C_FEAT),
        vals_f.reshape(B, M, K),
    )


# speedup vs baseline: 10.3180x; 1.1445x over previous
import functools

import jax
import jax.numpy as jnp
from jax import lax
from jax.experimental import pallas as pl
from jax.experimental.pallas import tpu as pltpu
from jax.experimental.pallas import tpu_sc as plsc

K = 32
B = 4
M = 2048
N = 8192
C_XYZ = 6
C_FEAT = 64
BM = 256

NW = 32
RPW = (B * M) // NW
ROWS_W = RPW * K
FCHUNK = 1024
NV = N // 16
CAP = 2048
REGN = 12
INF = float("inf")


def _dist_body(low_ref, high_ref, dist_ref):
    low = low_ref[0]
    high = high_ref[0]
    dx = low[:, 0:1] - high[:, 0][None, :]
    dy = low[:, 1:2] - high[:, 1][None, :]
    dz = low[:, 2:3] - high[:, 2][None, :]
    dist_ref[0] = jnp.sqrt((dx * dx + dy * dy) + dz * dz)


def _dist_map(new_xyz, xyz):
    return pl.pallas_call(
        _dist_body,
        grid=(B, M // BM),
        in_specs=[
            pl.BlockSpec((1, BM, C_XYZ), lambda b, m: (b, m, 0)),
            pl.BlockSpec((1, N, C_XYZ), lambda b, m: (b, 0, 0)),
        ],
        out_specs=pl.BlockSpec((1, BM, N), lambda b, m: (b, m, 0)),
        out_shape=jax.ShapeDtypeStruct((B, M, N), jnp.float32),
    )(new_xyz, xyz)


def _make_select():
    mesh = plsc.VectorSubcoreMesh(core_axis_name="c", subcore_axis_name="s")

    @functools.partial(
        pl.kernel,
        mesh=mesh,
        compiler_params=pltpu.CompilerParams(needs_layout_passes=False),
        out_type=[
            jax.ShapeDtypeStruct((B * M * K,), jnp.float32),
            jax.ShapeDtypeStruct((B * M * K,), jnp.int32),
            jax.ShapeDtypeStruct((B * M * K,), jnp.int32),
        ],
        scratch_types=[
            pltpu.VMEM((N,), jnp.float32),
            pltpu.VMEM((N,), jnp.float32),
            pltpu.VMEM((CAP + 16,), jnp.float32),
            pltpu.VMEM((CAP + 16,), jnp.int32),
            pltpu.VMEM((RPW * K,), jnp.float32),
            pltpu.VMEM((RPW * K,), jnp.int32),
            pltpu.VMEM((RPW * K,), jnp.int32),
            pltpu.SemaphoreType.DMA,
            pltpu.SemaphoreType.DMA,
        ],
    )
    def select(dist_hbm, vals_hbm, idx_hbm, idxabs_hbm,
               row0_v, row1_v, cval_v, cidx_v, ov_v, oi_v, oa_v, sem0, sem1):
        wid = lax.axis_index("s") * 2 + lax.axis_index("c")
        base = wid * RPW
        lane = lax.broadcasted_iota(jnp.int32, (16,), 0)
        rows = (row0_v, row1_v)
        sems = (sem0, sem1)

        pltpu.async_copy(dist_hbm.at[pl.ds(base * N, N)], row0_v, sem0)

        inf_spl = jnp.full((16,), INF)
        n_spl = jnp.full((16,), N, jnp.int32)

        def process(r, row_v):
            def gstep(h, carry):
                a0, a1 = carry
                for t in range(4):
                    a0 = jnp.minimum(a0, row_v[pl.ds((8 * h + 2 * t) * 16, 16)])
                    a1 = jnp.minimum(a1, row_v[pl.ds((8 * h + 2 * t + 1) * 16,
                                                     16)])
                return a0, a1

            acc0, acc1 = lax.fori_loop(0, NV // 8, gstep, (inf_spl, inf_spl))
            u = jnp.max(jnp.maximum(acc0, acc1))
            u_spl = jnp.full((16,), u, jnp.float32)

            for j in range(REGN):
                cval_v[pl.ds(j * 16, 16)] = inf_spl

            def cstep(h, basev):
                for t in range(4):
                    g = 4 * h + t
                    v = row_v[pl.ds(g * 16, 16)]
                    msk = v <= u_spl
                    mi = msk.astype(jnp.int32)
                    excl = plsc.cumsum(mi) - mi
                    dest = jnp.minimum(basev + excl, CAP)
                    plsc.store_scatter(cval_v, [dest], v, mask=msk)
                    plsc.store_scatter(cidx_v, [dest], g * 16 + lane, mask=msk)
                    basev = basev + plsc.all_reduce_population_count(msk)
                return basev

            basev = lax.fori_loop(0, NV // 4, cstep,
                                  jnp.zeros((16,), jnp.int32))
            off = jnp.minimum(jnp.max(basev), CAP)
            cval_v[pl.ds(off, 16)] = inf_spl
            nv = (off + 15) // 16

            boff = ((base + r) // M) * N
            z_f = jnp.zeros((16,), jnp.float32)
            z_i = jnp.zeros((16,), jnp.int32)

            def fast_path(_):
                cv = [cval_v[pl.ds(j * 16, 16)] for j in range(REGN)]
                ci = [cidx_v[pl.ds(j * 16, 16)] for j in range(REGN)]

                def kstep(k, carry):
                    av0, av1, ai0, ai1 = tuple(carry[:4])
                    cvs = list(carry[4:4 + REGN])
                    cis = list(carry[4 + REGN:])
                    mv, miv = cvs[0], cis[0]
                    for j in range(1, REGN):
                        sel = (cvs[j] < mv) | ((cvs[j] == mv) & (cis[j] < miv))
                        mv = jnp.where(sel, cvs[j], mv)
                        miv = jnp.where(sel, cis[j], miv)
                    m = jnp.min(mv)
                    m_spl = jnp.full((16,), m, jnp.float32)
                    i = jnp.min(jnp.where(mv == m_spl, miv, n_spl))
                    i_spl = jnp.full((16,), i, jnp.int32)
                    for j in range(REGN):
                        cvs[j] = jnp.where(cis[j] == i_spl, INF, cvs[j])
                    sel_lo = lane == (k % 16)
                    hi = k // 16
                    av0 = jnp.where(sel_lo & (hi == 0), m, av0)
                    av1 = jnp.where(sel_lo & (hi == 1), m, av1)
                    ai0 = jnp.where(sel_lo & (hi == 0), i, ai0)
                    ai1 = jnp.where(sel_lo & (hi == 1), i, ai1)
                    return tuple([av0, av1, ai0, ai1] + cvs + cis)

                out = lax.fori_loop(0, K, kstep,
                                    tuple([z_f, z_f, z_i, z_i] + cv + ci))
                return tuple(out[:4])

            def slow_path(_):
                def kstep(k, carry):
                    av0, av1, ai0, ai1 = carry

                    def mstep(g, a):
                        mv, miv = a
                        v = cval_v[pl.ds(g * 16, 16)]
                        iv = cidx_v[pl.ds(g * 16, 16)]
                        sel = (v < mv) | ((v == mv) & (iv < miv))
                        return jnp.where(sel, v, mv), jnp.where(sel, iv, miv)

                    mv, miv = lax.fori_loop(0, nv, mstep, (inf_spl, n_spl))
                    m = jnp.min(mv)
                    m_spl = jnp.full((16,), m, jnp.float32)
                    i = jnp.min(jnp.where(mv == m_spl, miv, n_spl))
                    i_spl = jnp.full((16,), i, jnp.int32)

                    def wstep(g, a):
                        v = cval_v[pl.ds(g * 16, 16)]
                        iv = cidx_v[pl.ds(g * 16, 16)]
                        cval_v[pl.ds(g * 16, 16)] = jnp.where(iv == i_spl,
                                                              INF, v)
                        return a

                    lax.fori_loop(0, nv, wstep, 0)
                    sel_lo = lane == (k % 16)
                    hi = k // 16
                    av0 = jnp.where(sel_lo & (hi == 0), m, av0)
                    av1 = jnp.where(sel_lo & (hi == 1), m, av1)
                    ai0 = jnp.where(sel_lo & (hi == 0), i, ai0)
                    ai1 = jnp.where(sel_lo & (hi == 1), i, ai1)
                    return av0, av1, ai0, ai1

                return lax.fori_loop(0, K, kstep, (z_f, z_f, z_i, z_i))

            av0, av1, ai0, ai1 = lax.cond(off <= REGN * 16,
                                          fast_path, slow_path, 0)
            ov_v[pl.ds(r * K, 16)] = av0
            ov_v[pl.ds(r * K + 16, 16)] = av1
            oi_v[pl.ds(r * K, 16)] = ai0
            oi_v[pl.ds(r * K + 16, 16)] = ai1
            oa_v[pl.ds(r * K, 16)] = ai0 + boff
            oa_v[pl.ds(r * K + 16, 16)] = ai1 + boff

        def pair(g2, carry):
            for bsel in range(2):
                r = g2 * 2 + bsel
                pltpu.make_async_copy(
                    dist_hbm.at[pl.ds((base + r) * N, N)],
                    rows[bsel], sems[bsel]).wait()

                @pl.when(r + 1 < RPW)
                def _():
                    pltpu.async_copy(
                        dist_hbm.at[pl.ds((base + r + 1) * N, N)],
                        rows[1 - bsel], sems[1 - bsel])

                process(r, rows[bsel])
            return carry

        lax.fori_loop(0, RPW // 2, pair, 0)
        pltpu.sync_copy(ov_v, vals_hbm.at[pl.ds(base * K, RPW * K)])
        pltpu.sync_copy(oi_v, idx_hbm.at[pl.ds(base * K, RPW * K)])
        pltpu.sync_copy(oa_v, idxabs_hbm.at[pl.ds(base * K, RPW * K)])

    return select


def _make_xyz_gather():
    mesh = plsc.VectorSubcoreMesh(core_axis_name="c", subcore_axis_name="s")

    @functools.partial(
        pl.kernel,
        mesh=mesh,
        compiler_params=pltpu.CompilerParams(needs_layout_passes=False),
        out_type=jax.ShapeDtypeStruct((NW * ROWS_W * C_XYZ,), jnp.float32),
        scratch_types=[
            pltpu.VMEM((N * C_XYZ,), jnp.float32),
            pltpu.VMEM((ROWS_W,), jnp.int32),
            pltpu.VMEM((ROWS_W * C_XYZ,), jnp.float32),
        ],
    )
    def xyz_gather(xyz_hbm, idx_hbm, out_hbm, table_v, idx_v, rows_v):
        wid = lax.axis_index("s") * 2 + lax.axis_index("c")
        batch = wid // (NW // B)
        base = wid * ROWS_W
        pltpu.sync_copy(xyz_hbm.at[batch], table_v)
        pltpu.sync_copy(idx_hbm.at[pl.ds(base, ROWS_W)], idx_v)
        lane = lax.broadcasted_iota(jnp.int32, (16,), 0)

        def body(g, carry):
            iv = idx_v[pl.ds(g * 16, 16)] * C_XYZ
            dst16 = (g * 16 + lane) * C_XYZ
            for c in range(C_XYZ):
                vals = plsc.load_gather(table_v, [iv + c])
                plsc.store_scatter(rows_v, [dst16 + c], vals)
            return carry

        lax.fori_loop(0, ROWS_W // 16, body, 0)
        pltpu.sync_copy(rows_v, out_hbm.at[pl.ds(base * C_XYZ, ROWS_W * C_XYZ)])

    return xyz_gather


def _make_feat_gather():
    mesh = plsc.VectorSubcoreMesh(core_axis_name="c", subcore_axis_name="s")

    @functools.partial(
        pl.kernel,
        mesh=mesh,
        compiler_params=pltpu.CompilerParams(
            needs_layout_passes=False, use_tc_tiling_on_sc=False),
        out_type=jax.ShapeDtypeStruct((NW * ROWS_W, C_FEAT), jnp.float32),
        scratch_types=[
            pltpu.VMEM((FCHUNK,), jnp.int32),
            pltpu.VMEM((FCHUNK, C_FEAT), jnp.float32),
            pltpu.SemaphoreType.DMA,
        ],
    )
    def feat_gather(feat_hbm, idxabs_hbm, out_hbm, idx_v, rows_v, sem):
        wid = lax.axis_index("s") * 2 + lax.axis_index("c")
        base = wid * ROWS_W
        for ch in range(ROWS_W // FCHUNK):
            off = base + ch * FCHUNK
            pltpu.sync_copy(idxabs_hbm.at[pl.ds(off, FCHUNK)], idx_v)
            pltpu.async_copy(feat_hbm.at[idx_v], rows_v, sem).wait()
            pltpu.sync_copy(rows_v, out_hbm.at[pl.ds(off, FCHUNK)])

    return feat_gather


def kernel(xyz, new_xyz, features):
    dist = _dist_map(new_xyz, xyz)
    vals_f, idx_f, idxabs_f = _make_select()(dist.reshape(B * M * N))
    feat_flat = features.reshape(B * N, C_FEAT)

    neighbor_xyz = _make_xyz_gather()(xyz.reshape(B, N * C_XYZ), idx_f)
    neighbor_feature = _make_feat_gather()(feat_flat, idxabs_f)

    return (
        neighbor_xyz.reshape(B, M, K, C_XYZ),
        idx_f.reshape(B, M, K),
        neighbor_feature.reshape(B, M, K, C_FEAT),
        vals_f.reshape(B, M, K),
    )


# speedup vs baseline: 16.8195x; 1.6301x over previous
import functools

import jax
import jax.numpy as jnp
from jax import lax
from jax.experimental import pallas as pl
from jax.experimental.pallas import tpu as pltpu
from jax.experimental.pallas import tpu_sc as plsc

K = 32
B = 4
M = 2048
N = 8192
C_XYZ = 6
C_FEAT = 64
BM = 256

NW = 32
RPW = (B * M) // NW
ROWS_W = RPW * K
FCHUNK = 1024
NV = N // 16
NBLK = 32
CAP = 2048
REGN = 8
INF = float("inf")


def _dist_body(low_ref, high_ref, dist_ref):
    low = low_ref[0]
    high = high_ref[0]
    dx = low[:, 0:1] - high[:, 0][None, :]
    dy = low[:, 1:2] - high[:, 1][None, :]
    dz = low[:, 2:3] - high[:, 2][None, :]
    dist_ref[0] = jnp.sqrt((dx * dx + dy * dy) + dz * dz)


def _dist_map(new_xyz, xyz):
    return pl.pallas_call(
        _dist_body,
        grid=(B, M // BM),
        in_specs=[
            pl.BlockSpec((1, BM, C_XYZ), lambda b, m: (b, m, 0)),
            pl.BlockSpec((1, N, C_XYZ), lambda b, m: (b, 0, 0)),
        ],
        out_specs=pl.BlockSpec((1, BM, N), lambda b, m: (b, m, 0)),
        out_shape=jax.ShapeDtypeStruct((B, M, N), jnp.float32),
    )(new_xyz, xyz)


def _make_select():
    mesh = plsc.VectorSubcoreMesh(core_axis_name="c", subcore_axis_name="s")

    @functools.partial(
        pl.kernel,
        mesh=mesh,
        compiler_params=pltpu.CompilerParams(needs_layout_passes=False),
        out_type=[
            jax.ShapeDtypeStruct((B * M * K,), jnp.float32),
            jax.ShapeDtypeStruct((B * M * K,), jnp.int32),
            jax.ShapeDtypeStruct((B * M * K,), jnp.int32),
        ],
        scratch_types=[
            pltpu.VMEM((N,), jnp.float32),
            pltpu.VMEM((N,), jnp.float32),
            pltpu.VMEM((NBLK * 16,), jnp.float32),
            pltpu.VMEM((NBLK * 16 + 16,), jnp.int32),
            pltpu.VMEM((CAP + 16,), jnp.float32),
            pltpu.VMEM((CAP + 16,), jnp.int32),
            pltpu.VMEM((RPW * K,), jnp.float32),
            pltpu.VMEM((RPW * K,), jnp.int32),
            pltpu.VMEM((RPW * K,), jnp.int32),
            pltpu.SemaphoreType.DMA,
            pltpu.SemaphoreType.DMA,
        ],
    )
    def select(dist_hbm, vals_hbm, idx_hbm, idxabs_hbm,
               row0_v, row1_v, minb_v, pair_v, cval_v, cidx_v,
               ov_v, oi_v, oa_v, sem0, sem1):
        wid = lax.axis_index("s") * 2 + lax.axis_index("c")
        base = wid * RPW
        lane = lax.broadcasted_iota(jnp.int32, (16,), 0)
        lane16 = lane * 16
        rows = (row0_v, row1_v)
        sems = (sem0, sem1)

        pltpu.async_copy(dist_hbm.at[pl.ds(base * N, N)], row0_v, sem0)

        inf_spl = jnp.full((16,), INF)
        n_spl = jnp.full((16,), N, jnp.int32)

        def process(r, row_v):
            def blockstep(j, carry):
                mn, m2 = carry
                acc = row_v[pl.ds(j * 256, 16)]
                for t in range(1, 16):
                    acc = jnp.minimum(acc, row_v[pl.ds(j * 256 + t * 16, 16)])
                minb_v[pl.ds(j * 16, 16)] = acc
                m2 = jnp.minimum(m2, jnp.maximum(mn, acc))
                mn = jnp.minimum(mn, acc)
                return mn, m2

            _, m2 = lax.fori_loop(0, NBLK, blockstep, (inf_spl, inf_spl))
            u = jnp.max(m2)
            u_spl = jnp.full((16,), u, jnp.float32)

            for j in range(REGN):
                cval_v[pl.ds(j * 16, 16)] = inf_spl

            def pstep(j, pb):
                mj = minb_v[pl.ds(j * 16, 16)]
                msk = mj <= u_spl
                mi = msk.astype(jnp.int32)
                excl = plsc.cumsum(mi) - mi
                plsc.store_scatter(pair_v, [pb + excl], j * 256 + lane,
                                   mask=msk)
                return pb + plsc.all_reduce_population_count(msk)

            pb = lax.fori_loop(0, NBLK, pstep, jnp.zeros((16,), jnp.int32))
            npairs = jnp.max(pb)

            def cstep(i, basev):
                bb = plsc.load_gather(pair_v, [jnp.full((16,), i, jnp.int32)])
                idxv = bb + lane16
                v = plsc.load_gather(row_v, [idxv])
                msk = v <= u_spl
                mi = msk.astype(jnp.int32)
                excl = plsc.cumsum(mi) - mi
                dest = jnp.minimum(basev + excl, CAP)
                plsc.store_scatter(cval_v, [dest], v, mask=msk)
                plsc.store_scatter(cidx_v, [dest], idxv, mask=msk)
                return basev + plsc.all_reduce_population_count(msk)

            basev = lax.fori_loop(0, npairs, cstep,
                                  jnp.zeros((16,), jnp.int32))
            off = jnp.minimum(jnp.max(basev), CAP)
            cval_v[pl.ds(off, 16)] = inf_spl
            nv = (off + 15) // 16

            boff = ((base + r) // M) * N
            z_f = jnp.zeros((16,), jnp.float32)
            z_i = jnp.zeros((16,), jnp.int32)

            def fast_path(_):
                cv = [cval_v[pl.ds(j * 16, 16)] for j in range(REGN)]
                ci = [cidx_v[pl.ds(j * 16, 16)] for j in range(REGN)]

                def kstep(k, carry):
                    av0, av1, ai0, ai1 = tuple(carry[:4])
                    cvs = list(carry[4:4 + REGN])
                    cis = list(carry[4 + REGN:])
                    mv, miv = cvs[0], cis[0]
                    for j in range(1, REGN):
                        sel = (cvs[j] < mv) | ((cvs[j] == mv) & (cis[j] < miv))
                        mv = jnp.where(sel, cvs[j], mv)
                        miv = jnp.where(sel, cis[j], miv)
                    m = jnp.min(mv)
                    m_spl = jnp.full((16,), m, jnp.float32)
                    i = jnp.min(jnp.where(mv == m_spl, miv, n_spl))
                    i_spl = jnp.full((16,), i, jnp.int32)
                    for j in range(REGN):
                        cvs[j] = jnp.where(cis[j] == i_spl, INF, cvs[j])
                    sel_lo = lane == (k % 16)
                    hi = k // 16
                    av0 = jnp.where(sel_lo & (hi == 0), m, av0)
                    av1 = jnp.where(sel_lo & (hi == 1), m, av1)
                    ai0 = jnp.where(sel_lo & (hi == 0), i, ai0)
                    ai1 = jnp.where(sel_lo & (hi == 1), i, ai1)
                    return tuple([av0, av1, ai0, ai1] + cvs + cis)

                out = lax.fori_loop(0, K, kstep,
                                    tuple([z_f, z_f, z_i, z_i] + cv + ci))
                return tuple(out[:4])

            def slow_path(_):
                def kstep(k, carry):
                    av0, av1, ai0, ai1 = carry

                    def mstep(g, a):
                        mv, miv = a
                        v = cval_v[pl.ds(g * 16, 16)]
                        iv = cidx_v[pl.ds(g * 16, 16)]
                        sel = (v < mv) | ((v == mv) & (iv < miv))
                        return jnp.where(sel, v, mv), jnp.where(sel, iv, miv)

                    mv, miv = lax.fori_loop(0, nv, mstep, (inf_spl, n_spl))
                    m = jnp.min(mv)
                    m_spl = jnp.full((16,), m, jnp.float32)
                    i = jnp.min(jnp.where(mv == m_spl, miv, n_spl))
                    i_spl = jnp.full((16,), i, jnp.int32)

                    def wstep(g, a):
                        v = cval_v[pl.ds(g * 16, 16)]
                        iv = cidx_v[pl.ds(g * 16, 16)]
                        cval_v[pl.ds(g * 16, 16)] = jnp.where(iv == i_spl,
                                                              INF, v)
                        return a

                    lax.fori_loop(0, nv, wstep, 0)
                    sel_lo = lane == (k % 16)
                    hi = k // 16
                    av0 = jnp.where(sel_lo & (hi == 0), m, av0)
                    av1 = jnp.where(sel_lo & (hi == 1), m, av1)
                    ai0 = jnp.where(sel_lo & (hi == 0), i, ai0)
                    ai1 = jnp.where(sel_lo & (hi == 1), i, ai1)
                    return av0, av1, ai0, ai1

                return lax.fori_loop(0, K, kstep, (z_f, z_f, z_i, z_i))

            av0, av1, ai0, ai1 = lax.cond(off <= REGN * 16,
                                          fast_path, slow_path, 0)
            ov_v[pl.ds(r * K, 16)] = av0
            ov_v[pl.ds(r * K + 16, 16)] = av1
            oi_v[pl.ds(r * K, 16)] = ai0
            oi_v[pl.ds(r * K + 16, 16)] = ai1
            oa_v[pl.ds(r * K, 16)] = ai0 + boff
            oa_v[pl.ds(r * K + 16, 16)] = ai1 + boff

        def pair(g2, carry):
            for bsel in range(2):
                r = g2 * 2 + bsel
                pltpu.make_async_copy(
                    dist_hbm.at[pl.ds((base + r) * N, N)],
                    rows[bsel], sems[bsel]).wait()

                @pl.when(r + 1 < RPW)
                def _():
                    pltpu.async_copy(
                        dist_hbm.at[pl.ds((base + r + 1) * N, N)],
                        rows[1 - bsel], sems[1 - bsel])

                process(r, rows[bsel])
            return carry

        lax.fori_loop(0, RPW // 2, pair, 0)
        pltpu.sync_copy(ov_v, vals_hbm.at[pl.ds(base * K, RPW * K)])
        pltpu.sync_copy(oi_v, idx_hbm.at[pl.ds(base * K, RPW * K)])
        pltpu.sync_copy(oa_v, idxabs_hbm.at[pl.ds(base * K, RPW * K)])

    return select


def _make_xyz_gather():
    mesh = plsc.VectorSubcoreMesh(core_axis_name="c", subcore_axis_name="s")

    @functools.partial(
        pl.kernel,
        mesh=mesh,
        compiler_params=pltpu.CompilerParams(needs_layout_passes=False),
        out_type=jax.ShapeDtypeStruct((NW * ROWS_W * C_XYZ,), jnp.float32),
        scratch_types=[
            pltpu.VMEM((N * C_XYZ,), jnp.float32),
            pltpu.VMEM((ROWS_W,), jnp.int32),
            pltpu.VMEM((ROWS_W * C_XYZ,), jnp.float32),
        ],
    )
    def xyz_gather(xyz_hbm, idx_hbm, out_hbm, table_v, idx_v, rows_v):
        wid = lax.axis_index("s") * 2 + lax.axis_index("c")
        batch = wid // (NW // B)
        base = wid * ROWS_W
        pltpu.sync_copy(xyz_hbm.at[batch], table_v)
        pltpu.sync_copy(idx_hbm.at[pl.ds(base, ROWS_W)], idx_v)
        lane = lax.broadcasted_iota(jnp.int32, (16,), 0)

        def body(g, carry):
            iv = idx_v[pl.ds(g * 16, 16)] * C_XYZ
            dst16 = (g * 16 + lane) * C_XYZ
            for c in range(C_XYZ):
                vals = plsc.load_gather(table_v, [iv + c])
                plsc.store_scatter(rows_v, [dst16 + c], vals)
            return carry

        lax.fori_loop(0, ROWS_W // 16, body, 0)
        pltpu.sync_copy(rows_v, out_hbm.at[pl.ds(base * C_XYZ, ROWS_W * C_XYZ)])

    return xyz_gather


def _make_feat_gather():
    mesh = plsc.VectorSubcoreMesh(core_axis_name="c", subcore_axis_name="s")

    @functools.partial(
        pl.kernel,
        mesh=mesh,
        compiler_params=pltpu.CompilerParams(
            needs_layout_passes=False, use_tc_tiling_on_sc=False),
        out_type=jax.ShapeDtypeStruct((NW * ROWS_W, C_FEAT), jnp.float32),
        scratch_types=[
            pltpu.VMEM((FCHUNK,), jnp.int32),
            pltpu.VMEM((FCHUNK, C_FEAT), jnp.float32),
            pltpu.SemaphoreType.DMA,
        ],
    )
    def feat_gather(feat_hbm, idxabs_hbm, out_hbm, idx_v, rows_v, sem):
        wid = lax.axis_index("s") * 2 + lax.axis_index("c")
        base = wid * ROWS_W
        for ch in range(ROWS_W // FCHUNK):
            off = base + ch * FCHUNK
            pltpu.sync_copy(idxabs_hbm.at[pl.ds(off, FCHUNK)], idx_v)
            pltpu.async_copy(feat_hbm.at[idx_v], rows_v, sem).wait()
            pltpu.sync_copy(rows_v, out_hbm.at[pl.ds(off, FCHUNK)])

    return feat_gather


def kernel(xyz, new_xyz, features):
    dist = _dist_map(new_xyz, xyz)
    vals_f, idx_f, idxabs_f = _make_select()(dist.reshape(B * M * N))
    feat_flat = features.reshape(B * N, C_FEAT)

    neighbor_xyz = _make_xyz_gather()(xyz.reshape(B, N * C_XYZ), idx_f)
    neighbor_feature = _make_feat_gather()(feat_flat, idxabs_f)

    return (
        neighbor_xyz.reshape(B, M, K, C_XYZ),
        idx_f.reshape(B, M, K),
        neighbor_feature.reshape(B, M, K, C_FEAT),
        vals_f.reshape(B, M, K),
    )
